# baseline (speedup 1.0000x reference)
import functools

import jax
import jax.numpy as jnp
from jax import lax
from jax.experimental import pallas as pl
from jax.experimental.pallas import tpu as pltpu
from jax.experimental.pallas import tpu_sc as plsc

_BATCH = 16384
_FIELDS = 100
_FPAD = 112
_DIM = 64
_NC = 2
_NS = 16
_NW = _NC * _NS
_BPW = _BATCH // _NW
_KS = _FPAD // 16


def _bip_sc(x_pad, emb):
  mesh = plsc.VectorSubcoreMesh(core_axis_name="c", subcore_axis_name="s")

  @functools.partial(
      pl.kernel,
      out_type=jax.ShapeDtypeStruct((_BATCH, _DIM), jnp.float32),
      mesh=mesh,
      scratch_types=(
          [pltpu.VMEM((_BPW * _FPAD,), jnp.int32)]
          + [pltpu.VMEM((_FPAD, _DIM), jnp.float32)
             for _ in range(2)]
          + [pltpu.VMEM((_BPW, _DIM), jnp.float32)]
          + [pltpu.SemaphoreType.DMA for _ in range(2)]
      ),
      compiler_params=pltpu.CompilerParams(use_tc_tiling_on_sc=False),
  )
  def k(x_hbm, emb_hbm, h_hbm, idx_v, rows0, rows1, out_v, sem0, sem1):
    rows_bufs = (rows0, rows1)
    sems = (sem0, sem1)
    wid = lax.axis_index("s") * _NC + lax.axis_index("c")
    base = wid * _BPW
    pltpu.sync_copy(x_hbm.at[wid], idx_v)

    def fire(i, b):
      for kk in range(_KS):
        iv = idx_v[pl.ds(i * _FPAD + kk * 16, 16)]
        pltpu.make_async_copy(
            emb_hbm.at[iv], rows_bufs[b].at[pl.ds(kk * 16, 16)],
            sems[b]).start()

    def drain(b):
      for kk in range(_KS):
        pltpu.make_async_copy(
            emb_hbm.at[pl.ds(0, 16)], rows_bufs[b].at[pl.ds(kk * 16, 16)],
            sems[b]).wait()

    def process(i, b):
      rows = rows_bufs[b]
      zero = jnp.zeros((16,), jnp.float32)

      def body(f, carry):
        s0, s1, s2, s3, q0, q1, q2, q3 = carry
        v0 = rows[f, pl.ds(0, 16)]
        v1 = rows[f, pl.ds(16, 16)]
        v2 = rows[f, pl.ds(32, 16)]
        v3 = rows[f, pl.ds(48, 16)]
        return (s0 + v0, s1 + v1, s2 + v2, s3 + v3,
                q0 + v0 * v0, q1 + v1 * v1, q2 + v2 * v2, q3 + v3 * v3)

      acc = lax.fori_loop(0, _FIELDS, body, (zero,) * 8, unroll=4)
      for c in range(4):
        sm, q = acc[c], acc[4 + c]
        out_v[i, pl.ds(c * 16, 16)] = (sm * sm - q) * 0.5

    fire(0, 0)
    fire(1, 1)

    def step(j, carry):
      i0 = 2 * j
      for b in range(2):
        drain(b)
        process(i0 + b, b)
        fire(i0 + b + 2, b)
      return carry

    lax.fori_loop(0, _BPW // 2 - 1, step, 0)
    for b in range(2):
      drain(b)
      process(_BPW - 2 + b, b)
    pltpu.sync_copy(out_v, h_hbm.at[pl.ds(base, _BPW)])

  return k(x_pad, emb)


def _mlp_tc(h, w1t, b1, w2t, b2, wf, bf):
  blk = 1024

  def body(h_ref, w1_ref, b1_ref, w2_ref, b2_ref, wf_ref, bf_ref, o_ref):
    hb = h_ref[...]
    a1 = jnp.maximum(
        jnp.dot(hb, w1_ref[...], preferred_element_type=jnp.float32)
        + b1_ref[...], 0.0)
    a2 = jnp.maximum(
        jnp.dot(a1, w2_ref[...], preferred_element_type=jnp.float32)
        + b2_ref[...], 0.0)
    t = jnp.sum(a2 * wf_ref[...], axis=1, keepdims=True) + bf_ref[...]
    o_ref[...] = 1.0 / (1.0 + jnp.exp(-t))

  return pl.pallas_call(
      body,
      grid=(_BATCH // blk,),
      in_specs=[
          pl.BlockSpec((blk, _DIM), lambda i: (i, 0)),
          pl.BlockSpec((_DIM, 32), lambda i: (0, 0)),
          pl.BlockSpec((1, 32), lambda i: (0, 0)),
          pl.BlockSpec((32, 16), lambda i: (0, 0)),
          pl.BlockSpec((1, 16), lambda i: (0, 0)),
          pl.BlockSpec((1, 16), lambda i: (0, 0)),
          pl.BlockSpec((1, 1), lambda i: (0, 0)),
      ],
      out_specs=pl.BlockSpec((blk, 1), lambda i: (i, 0)),
      out_shape=jax.ShapeDtypeStruct((_BATCH, 1), jnp.float32),
  )(h, w1t, b1, w2t, b2, wf, bf)


def kernel(x, Emb, W1, b1, W2, b2, Wf, bf):
  x = x.astype(jnp.int32)
  xp = jnp.pad(x, ((0, 0), (0, _FPAD - _FIELDS)))
  xw = xp.reshape(_NW, _BPW * _FPAD)
  h = _bip_sc(xw, Emb)
  return _mlp_tc(h, W1.T, b1.reshape(1, -1), W2.T, b2.reshape(1, -1),
                 Wf, bf.reshape(1, 1))


# speedup vs baseline: 4.8150x; 4.8150x over previous
import functools

import jax
import jax.numpy as jnp
from jax import lax
from jax.experimental import pallas as pl
from jax.experimental.pallas import tpu as pltpu
from jax.experimental.pallas import tpu_sc as plsc

_BATCH = 16384
_FIELDS = 100
_FPAD = 100
_DIM = 64
_NC = 2
_NS = 16
_NW = _NC * _NS
_BPW = _BATCH // _NW


def _bip_sc(x_pad, emb):
  mesh = plsc.VectorSubcoreMesh(core_axis_name="c", subcore_axis_name="s")

  @functools.partial(
      pl.kernel,
      out_type=jax.ShapeDtypeStruct((_BATCH, _DIM), jnp.float32),
      mesh=mesh,
      scratch_types=(
          [pltpu.VMEM((_BPW, _FPAD), jnp.int32)]
          + [pltpu.VMEM((_FPAD, _DIM), jnp.float32)
             for _ in range(2)]
          + [pltpu.VMEM((_BPW, _DIM), jnp.float32)]
          + [pltpu.SemaphoreType.DMA for _ in range(2)]
      ),
      compiler_params=pltpu.CompilerParams(use_tc_tiling_on_sc=False),
  )
  def k(x_hbm, emb_hbm, h_hbm, idx_v, rows0, rows1, out_v, sem0, sem1):
    rows_bufs = (rows0, rows1)
    sems = (sem0, sem1)
    wid = lax.axis_index("s") * _NC + lax.axis_index("c")
    base = wid * _BPW
    pltpu.sync_copy(x_hbm.at[pl.ds(base, _BPW)], idx_v)

    def fire(i, b):
      pltpu.make_async_copy(
          emb_hbm.at[idx_v.at[i]], rows_bufs[b],
          sems[b]).start()

    def drain(b):
      pltpu.make_async_copy(
          emb_hbm.at[idx_v.at[0]], rows_bufs[b],
          sems[b]).wait()

    def process(i, b):
      rows = rows_bufs[b]
      zero = jnp.zeros((16,), jnp.float32)

      def body(f, carry):
        s0, s1, s2, s3, q0, q1, q2, q3 = carry
        v0 = rows[f, pl.ds(0, 16)]
        v1 = rows[f, pl.ds(16, 16)]
        v2 = rows[f, pl.ds(32, 16)]
        v3 = rows[f, pl.ds(48, 16)]
        return (s0 + v0, s1 + v1, s2 + v2, s3 + v3,
                q0 + v0 * v0, q1 + v1 * v1, q2 + v2 * v2, q3 + v3 * v3)

      acc = lax.fori_loop(0, _FIELDS, body, (zero,) * 8, unroll=4)
      for c in range(4):
        sm, q = acc[c], acc[4 + c]
        out_v[i, pl.ds(c * 16, 16)] = (sm * sm - q) * 0.5

    fire(0, 0)
    fire(1, 1)

    def step(j, carry):
      i0 = 2 * j
      for b in range(2):
        drain(b)
        process(i0 + b, b)
        fire(i0 + b + 2, b)
      return carry

    lax.fori_loop(0, _BPW // 2 - 1, step, 0)
    for b in range(2):
      drain(b)
      process(_BPW - 2 + b, b)
    pltpu.sync_copy(out_v, h_hbm.at[pl.ds(base, _BPW)])

  return k(x_pad, emb)


def _mlp_tc(h, w1t, b1, w2t, b2, wf, bf):
  blk = 1024

  def body(h_ref, w1_ref, b1_ref, w2_ref, b2_ref, wf_ref, bf_ref, o_ref):
    dn = (((1,), (1,)), ((), ()))
    hb = h_ref[...]
    a1 = jnp.maximum(
        lax.dot_general(hb, w1_ref[...], dn,
                        preferred_element_type=jnp.float32)
        + b1_ref[...], 0.0)
    a2 = jnp.maximum(
        lax.dot_general(a1, w2_ref[...], dn,
                        preferred_element_type=jnp.float32)
        + b2_ref[...], 0.0)
    t = jnp.sum(a2 * wf_ref[...], axis=1, keepdims=True) + bf_ref[...]
    o_ref[...] = 1.0 / (1.0 + jnp.exp(-t))

  return pl.pallas_call(
      body,
      grid=(_BATCH // blk,),
      in_specs=[
          pl.BlockSpec((blk, _DIM), lambda i: (i, 0)),
          pl.BlockSpec((32, _DIM), lambda i: (0, 0)),
          pl.BlockSpec((1, 32), lambda i: (0, 0)),
          pl.BlockSpec((16, 32), lambda i: (0, 0)),
          pl.BlockSpec((1, 16), lambda i: (0, 0)),
          pl.BlockSpec((1, 16), lambda i: (0, 0)),
          pl.BlockSpec((1, 1), lambda i: (0, 0)),
      ],
      out_specs=pl.BlockSpec((blk, 1), lambda i: (i, 0)),
      out_shape=jax.ShapeDtypeStruct((_BATCH, 1), jnp.float32),
  )(h, w1t, b1, w2t, b2, wf, bf)


def kernel(x, Emb, W1, b1, W2, b2, Wf, bf):
  h = _bip_sc(x.astype(jnp.int32), Emb)
  return _mlp_tc(h, W1, b1.reshape(1, -1), W2, b2.reshape(1, -1),
                 Wf, bf.reshape(1, 1))


# speedup vs baseline: 5.3075x; 1.1023x over previous
import functools

import jax
import jax.numpy as jnp
from jax import lax
from jax.experimental import pallas as pl
from jax.experimental.pallas import tpu as pltpu
from jax.experimental.pallas import tpu_sc as plsc

_BATCH = 16384
_FIELDS = 100
_FPAD = 100
_DIM = 64
_NC = 2
_NS = 16
_NW = _NC * _NS
_BPW = _BATCH // _NW


def _bip_sc(x_pad, emb):
  mesh = plsc.VectorSubcoreMesh(core_axis_name="c", subcore_axis_name="s")

  @functools.partial(
      pl.kernel,
      out_type=jax.ShapeDtypeStruct((_BATCH, _DIM), jnp.float32),
      mesh=mesh,
      scratch_types=(
          [pltpu.VMEM((_BPW, _FPAD), jnp.int32)]
          + [pltpu.VMEM((_FPAD, _DIM), jnp.float32)
             for _ in range(3)]
          + [pltpu.VMEM((_BPW, _DIM), jnp.float32)]
          + [pltpu.SemaphoreType.DMA for _ in range(3)]
      ),
      compiler_params=pltpu.CompilerParams(use_tc_tiling_on_sc=False),
  )
  def k(x_hbm, emb_hbm, h_hbm, idx_v, rows0, rows1, rows2, out_v,
        sem0, sem1, sem2):
    rows_bufs = (rows0, rows1, rows2)
    sems = (sem0, sem1, sem2)
    wid = lax.axis_index("s") * _NC + lax.axis_index("c")
    base = wid * _BPW
    pltpu.sync_copy(x_hbm.at[pl.ds(base, _BPW)], idx_v)

    def fire(i, b):
      pltpu.make_async_copy(
          emb_hbm.at[idx_v.at[i]], rows_bufs[b],
          sems[b]).start()

    def drain(b):
      pltpu.make_async_copy(
          emb_hbm.at[idx_v.at[0]], rows_bufs[b],
          sems[b]).wait()

    def process(i, b):
      rows = rows_bufs[b]
      zero = jnp.zeros((16,), jnp.float32)

      def body(f, carry):
        s0, s1, s2, s3, q0, q1, q2, q3 = carry
        v0 = rows[f, pl.ds(0, 16)]
        v1 = rows[f, pl.ds(16, 16)]
        v2 = rows[f, pl.ds(32, 16)]
        v3 = rows[f, pl.ds(48, 16)]
        return (s0 + v0, s1 + v1, s2 + v2, s3 + v3,
                q0 + v0 * v0, q1 + v1 * v1, q2 + v2 * v2, q3 + v3 * v3)

      acc = lax.fori_loop(0, _FIELDS, body, (zero,) * 8, unroll=4)
      for c in range(4):
        sm, q = acc[c], acc[4 + c]
        out_v[i, pl.ds(c * 16, 16)] = (sm * sm - q) * 0.5

    fire(0, 0)
    fire(1, 1)
    fire(2, 2)

    def step(j, carry):
      i0 = 3 * j
      for b in range(3):
        drain(b)
        process(i0 + b, b)
        fire(i0 + b + 3, b)
      return carry

    lax.fori_loop(0, _BPW // 3 - 1, step, 0)
    i0 = 3 * (_BPW // 3 - 1)
    for t, b in enumerate((0, 1, 2, 0, 1)):
      drain(b)
      process(i0 + t, b)
      if t + i0 + 3 < _BPW:
        fire(i0 + t + 3, b)
    pltpu.sync_copy(out_v, h_hbm.at[pl.ds(base, _BPW)])

  return k(x_pad, emb)


def _mlp_tc(h, w1t, b1, w2t, b2, wf, bf):
  blk = 1024

  def body(h_ref, w1_ref, b1_ref, w2_ref, b2_ref, wf_ref, bf_ref, o_ref):
    dn = (((1,), (1,)), ((), ()))
    hb = h_ref[...]
    a1 = jnp.maximum(
        lax.dot_general(hb, w1_ref[...], dn,
                        preferred_element_type=jnp.float32)
        + b1_ref[...], 0.0)
    a2 = jnp.maximum(
        lax.dot_general(a1, w2_ref[...], dn,
                        preferred_element_type=jnp.float32)
        + b2_ref[...], 0.0)
    t = jnp.sum(a2 * wf_ref[...], axis=1, keepdims=True) + bf_ref[...]
    o_ref[...] = 1.0 / (1.0 + jnp.exp(-t))

  return pl.pallas_call(
      body,
      grid=(_BATCH // blk,),
      in_specs=[
          pl.BlockSpec((blk, _DIM), lambda i: (i, 0)),
          pl.BlockSpec((32, _DIM), lambda i: (0, 0)),
          pl.BlockSpec((1, 32), lambda i: (0, 0)),
          pl.BlockSpec((16, 32), lambda i: (0, 0)),
          pl.BlockSpec((1, 16), lambda i: (0, 0)),
          pl.BlockSpec((1, 16), lambda i: (0, 0)),
          pl.BlockSpec((1, 1), lambda i: (0, 0)),
      ],
      out_specs=pl.BlockSpec((blk, 1), lambda i: (i, 0)),
      out_shape=jax.ShapeDtypeStruct((_BATCH, 1), jnp.float32),
  )(h, w1t, b1, w2t, b2, wf, bf)


def kernel(x, Emb, W1, b1, W2, b2, Wf, bf):
  h = _bip_sc(x.astype(jnp.int32), Emb)
  return _mlp_tc(h, W1, b1.reshape(1, -1), W2, b2.reshape(1, -1),
                 Wf, bf.reshape(1, 1))


# speedup vs baseline: 5.5681x; 1.0491x over previous
import functools

import jax
import jax.numpy as jnp
from jax import lax
from jax.experimental import pallas as pl
from jax.experimental.pallas import tpu as pltpu
from jax.experimental.pallas import tpu_sc as plsc

_BATCH = 16384
_FIELDS = 100
_FPAD = 100
_DIM = 64
_NC = 2
_NS = 16
_NW = _NC * _NS
_BPW = _BATCH // _NW


def _bip_sc(x_pad, emb):
  mesh = plsc.VectorSubcoreMesh(core_axis_name="c", subcore_axis_name="s")

  @functools.partial(
      pl.kernel,
      out_type=jax.ShapeDtypeStruct((_BATCH, _DIM), jnp.float32),
      mesh=mesh,
      scratch_types=(
          [pltpu.VMEM((_BPW, _FPAD), jnp.int32)]
          + [pltpu.VMEM((_FPAD, _DIM), jnp.float32)
             for _ in range(4)]
          + [pltpu.VMEM((_BPW, _DIM), jnp.float32)]
          + [pltpu.SemaphoreType.DMA for _ in range(4)]
      ),
      compiler_params=pltpu.CompilerParams(use_tc_tiling_on_sc=False),
  )
  def k(x_hbm, emb_hbm, h_hbm, idx_v, rows0, rows1, rows2, rows3, out_v,
        sem0, sem1, sem2, sem3):
    rows_bufs = (rows0, rows1, rows2, rows3)
    sems = (sem0, sem1, sem2, sem3)
    wid = lax.axis_index("s") * _NC + lax.axis_index("c")
    base = wid * _BPW
    pltpu.sync_copy(x_hbm.at[pl.ds(base, _BPW)], idx_v)

    def fire(i, b):
      pltpu.make_async_copy(
          emb_hbm.at[idx_v.at[i]], rows_bufs[b],
          sems[b]).start()

    def drain(b):
      pltpu.make_async_copy(
          emb_hbm.at[idx_v.at[0]], rows_bufs[b],
          sems[b]).wait()

    def process(i, b):
      rows = rows_bufs[b]
      zero = jnp.zeros((16,), jnp.float32)

      def body(f, carry):
        s0, s1, s2, s3, q0, q1, q2, q3 = carry
        v0 = rows[f, pl.ds(0, 16)]
        v1 = rows[f, pl.ds(16, 16)]
        v2 = rows[f, pl.ds(32, 16)]
        v3 = rows[f, pl.ds(48, 16)]
        return (s0 + v0, s1 + v1, s2 + v2, s3 + v3,
                q0 + v0 * v0, q1 + v1 * v1, q2 + v2 * v2, q3 + v3 * v3)

      acc = lax.fori_loop(0, _FIELDS, body, (zero,) * 8, unroll=4)
      for c in range(4):
        sm, q = acc[c], acc[4 + c]
        out_v[i, pl.ds(c * 16, 16)] = (sm * sm - q) * 0.5

    for b in range(4):
      fire(b, b)

    def step(j, carry):
      i0 = 4 * j
      for b in range(4):
        drain(b)
        process(i0 + b, b)
        fire(i0 + b + 4, b)
      return carry

    lax.fori_loop(0, _BPW // 4 - 1, step, 0)
    for b in range(4):
      drain(b)
      process(_BPW - 4 + b, b)
    pltpu.sync_copy(out_v, h_hbm.at[pl.ds(base, _BPW)])

  return k(x_pad, emb)


def _mlp_tc(h, w1t, b1, w2t, b2, wf, bf):
  blk = 1024

  def body(h_ref, w1_ref, b1_ref, w2_ref, b2_ref, wf_ref, bf_ref, o_ref):
    dn = (((1,), (1,)), ((), ()))
    hb = h_ref[...]
    a1 = jnp.maximum(
        lax.dot_general(hb, w1_ref[...], dn,
                        preferred_element_type=jnp.float32)
        + b1_ref[...], 0.0)
    a2 = jnp.maximum(
        lax.dot_general(a1, w2_ref[...], dn,
                        preferred_element_type=jnp.float32)
        + b2_ref[...], 0.0)
    t = jnp.sum(a2 * wf_ref[...], axis=1, keepdims=True) + bf_ref[...]
    o_ref[...] = 1.0 / (1.0 + jnp.exp(-t))

  return pl.pallas_call(
      body,
      grid=(_BATCH // blk,),
      in_specs=[
          pl.BlockSpec((blk, _DIM), lambda i: (i, 0)),
          pl.BlockSpec((32, _DIM), lambda i: (0, 0)),
          pl.BlockSpec((1, 32), lambda i: (0, 0)),
          pl.BlockSpec((16, 32), lambda i: (0, 0)),
          pl.BlockSpec((1, 16), lambda i: (0, 0)),
          pl.BlockSpec((1, 16), lambda i: (0, 0)),
          pl.BlockSpec((1, 1), lambda i: (0, 0)),
      ],
      out_specs=pl.BlockSpec((blk, 1), lambda i: (i, 0)),
      out_shape=jax.ShapeDtypeStruct((_BATCH, 1), jnp.float32),
  )(h, w1t, b1, w2t, b2, wf, bf)


def kernel(x, Emb, W1, b1, W2, b2, Wf, bf):
  h = _bip_sc(x.astype(jnp.int32), Emb)
  return _mlp_tc(h, W1, b1.reshape(1, -1), W2, b2.reshape(1, -1),
                 Wf, bf.reshape(1, 1))


# speedup vs baseline: 5.7724x; 1.0367x over previous
import functools

import jax
import jax.numpy as jnp
from jax import lax
from jax.experimental import pallas as pl
from jax.experimental.pallas import tpu as pltpu
from jax.experimental.pallas import tpu_sc as plsc

_BATCH = 16384
_FIELDS = 100
_FPAD = 100
_DIM = 64
_NC = 2
_NS = 16
_NW = _NC * _NS
_BPW = _BATCH // _NW


def _bip_sc(x_pad, emb):
  mesh = plsc.VectorSubcoreMesh(core_axis_name="c", subcore_axis_name="s")

  @functools.partial(
      pl.kernel,
      out_type=jax.ShapeDtypeStruct((_BATCH, _DIM), jnp.float32),
      mesh=mesh,
      scratch_types=(
          [pltpu.VMEM((_BPW, _FPAD), jnp.int32)]
          + [pltpu.VMEM((_FPAD, _DIM), jnp.float32)
             for _ in range(6)]
          + [pltpu.VMEM((_BPW, _DIM), jnp.float32)]
          + [pltpu.SemaphoreType.DMA for _ in range(6)]
      ),
      compiler_params=pltpu.CompilerParams(use_tc_tiling_on_sc=False),
  )
  def k(x_hbm, emb_hbm, h_hbm, idx_v, rows0, rows1, rows2, rows3, rows4,
        rows5, out_v, sem0, sem1, sem2, sem3, sem4, sem5):
    rows_bufs = (rows0, rows1, rows2, rows3, rows4, rows5)
    sems = (sem0, sem1, sem2, sem3, sem4, sem5)
    wid = lax.axis_index("s") * _NC + lax.axis_index("c")
    base = wid * _BPW
    pltpu.sync_copy(x_hbm.at[pl.ds(base, _BPW)], idx_v)

    def fire(i, b):
      pltpu.make_async_copy(
          emb_hbm.at[idx_v.at[i]], rows_bufs[b],
          sems[b]).start()

    def drain(b):
      pltpu.make_async_copy(
          emb_hbm.at[idx_v.at[0]], rows_bufs[b],
          sems[b]).wait()

    def process(i, b):
      rows = rows_bufs[b]
      zero = jnp.zeros((16,), jnp.float32)

      def body(f, carry):
        s0, s1, s2, s3, q0, q1, q2, q3 = carry
        v0 = rows[f, pl.ds(0, 16)]
        v1 = rows[f, pl.ds(16, 16)]
        v2 = rows[f, pl.ds(32, 16)]
        v3 = rows[f, pl.ds(48, 16)]
        return (s0 + v0, s1 + v1, s2 + v2, s3 + v3,
                q0 + v0 * v0, q1 + v1 * v1, q2 + v2 * v2, q3 + v3 * v3)

      acc = lax.fori_loop(0, _FIELDS, body, (zero,) * 8, unroll=4)
      for c in range(4):
        sm, q = acc[c], acc[4 + c]
        out_v[i, pl.ds(c * 16, 16)] = (sm * sm - q) * 0.5

    for b in range(6):
      fire(b, b)

    def step(j, carry):
      i0 = 6 * j
      for b in range(6):
        drain(b)
        process(i0 + b, b)
        fire(i0 + b + 6, b)
      return carry

    lax.fori_loop(0, _BPW // 6 - 1, step, 0)
    i0 = 6 * (_BPW // 6 - 1)
    for t in range(8):
      b = t % 6
      drain(b)
      process(i0 + t, b)
      if i0 + t + 6 < _BPW:
        fire(i0 + t + 6, b)
    pltpu.sync_copy(out_v, h_hbm.at[pl.ds(base, _BPW)])

  return k(x_pad, emb)


def _mlp_tc(h, w1t, b1, w2t, b2, wf, bf):
  blk = 1024

  def body(h_ref, w1_ref, b1_ref, w2_ref, b2_ref, wf_ref, bf_ref, o_ref):
    dn = (((1,), (1,)), ((), ()))
    hb = h_ref[...]
    a1 = jnp.maximum(
        lax.dot_general(hb, w1_ref[...], dn,
                        preferred_element_type=jnp.float32)
        + b1_ref[...], 0.0)
    a2 = jnp.maximum(
        lax.dot_general(a1, w2_ref[...], dn,
                        preferred_element_type=jnp.float32)
        + b2_ref[...], 0.0)
    t = jnp.sum(a2 * wf_ref[...], axis=1, keepdims=True) + bf_ref[...]
    o_ref[...] = 1.0 / (1.0 + jnp.exp(-t))

  return pl.pallas_call(
      body,
      grid=(_BATCH // blk,),
      in_specs=[
          pl.BlockSpec((blk, _DIM), lambda i: (i, 0)),
          pl.BlockSpec((32, _DIM), lambda i: (0, 0)),
          pl.BlockSpec((1, 32), lambda i: (0, 0)),
          pl.BlockSpec((16, 32), lambda i: (0, 0)),
          pl.BlockSpec((1, 16), lambda i: (0, 0)),
          pl.BlockSpec((1, 16), lambda i: (0, 0)),
          pl.BlockSpec((1, 1), lambda i: (0, 0)),
      ],
      out_specs=pl.BlockSpec((blk, 1), lambda i: (i, 0)),
      out_shape=jax.ShapeDtypeStruct((_BATCH, 1), jnp.float32),
  )(h, w1t, b1, w2t, b2, wf, bf)


def kernel(x, Emb, W1, b1, W2, b2, Wf, bf):
  h = _bip_sc(x.astype(jnp.int32), Emb)
  return _mlp_tc(h, W1, b1.reshape(1, -1), W2, b2.reshape(1, -1),
                 Wf, bf.reshape(1, 1))
